# Initial kernel scaffold; baseline (speedup 1.0000x reference)
import jax
import jax.numpy as jnp
from jax.experimental import pallas as pl


def kernel(cell_x, cell_edge_index, tissue_x, tissue_edge_index, assignment_mat, W_rel, b_rel, W_root, gn_weight, gn_bias, gn_mean_scale, bn_gamma, bn_beta, lin_W, lin_b):
    raise NotImplementedError("write your pallas kernel here")


# speedup vs baseline: 4.1931x; 4.1931x over previous
import functools

import jax
import jax.numpy as jnp
from jax import lax
from jax.experimental import pallas as pl
from jax.experimental.pallas import tpu as pltpu
from jax.experimental.pallas import tpu_sc as plsc

N = 10000
E = 160000
D = 514
DOUT = 256
NPAD = 10240
CW = 144
NCHUNK = 4
DPAD = NCHUNK * CW

NC, NS = 2, 16
EPT = E // NS
WIN = 128
NWIN = 80
EPT_PAD = NWIN * WIN
ROWS_PER_TILE = NPAD // NS

NBLK = 25
BLK = N // NBLK


def _sc_segment_sum_body(t0, t1, t2, t3, edges3,
                         o0, o1, o2, o3,
                         spm, rows0, rows1, exr, gsem0, gsem1, isem):
    c = lax.axis_index("c")
    s = lax.axis_index("s")

    def load_idx(w, slot):
        pltpu.async_copy(edges3.at[s, w], exr.at[slot], isem)

    def wait_idx():
        pltpu.make_async_copy(edges3.at[s, 0], exr.at[0], isem).wait()

    def do_chunk(tab, out):
        def _zrow(r, _):
            def _zcol(k, _):
                rows0[r, pl.ds(k * 16, 16)] = jnp.zeros((16,), jnp.float32)
                return 0
            return lax.fori_loop(0, CW // 16, _zcol, 0)
        lax.fori_loop(0, WIN, _zrow, 0)
        for k in range(ROWS_PER_TILE // WIN):
            pltpu.sync_copy(rows0, spm.at[pl.ds(s * ROWS_PER_TILE + k * WIN, WIN)])
        plsc.subcore_barrier()

        def start(rbuf, slot, sem):
            pltpu.async_copy(tab.at[exr.at[slot, 0]], rbuf, sem)

        def wait(rbuf, sem):
            pltpu.make_async_copy(tab.at[exr.at[0, 0]], rbuf, sem).wait()

        def scatter(rbuf, slot):
            pltpu.sync_copy(rbuf, spm.at[exr.at[slot, 1]], add=True)

        pltpu.sync_copy(edges3.at[s, 0], exr.at[0])
        start(rows0, 0, gsem0)
        load_idx(1, 1)

        def gbody(g, _):
            b = 2 * g + 2
            wait_idx()
            start(rows1, 1, gsem1)
            wait(rows0, gsem0)
            scatter(rows0, 0)
            load_idx(b, 0)
            wait_idx()
            start(rows0, 0, gsem0)
            wait(rows1, gsem1)
            scatter(rows1, 1)
            load_idx(b + 1, 1)
            return 0
        lax.fori_loop(0, NWIN // 2 - 1, gbody, 0)
        wait_idx()
        start(rows1, 1, gsem1)
        wait(rows0, gsem0)
        scatter(rows0, 0)
        wait(rows1, gsem1)
        scatter(rows1, 1)

        plsc.subcore_barrier()
        pltpu.sync_copy(spm.at[pl.ds(s * ROWS_PER_TILE, ROWS_PER_TILE)],
                        out.at[pl.ds(s * ROWS_PER_TILE, ROWS_PER_TILE)])
        plsc.subcore_barrier()

    @pl.when(c == 0)
    def _():
        do_chunk(t0, o0)
        do_chunk(t1, o1)

    @pl.when(c == 1)
    def _():
        do_chunk(t2, o2)
        do_chunk(t3, o3)


def _sc_segment_sum(tables, edges3):
    mesh = plsc.VectorSubcoreMesh(core_axis_name="c", subcore_axis_name="s",
                                  num_cores=NC, num_subcores=NS)
    f = pl.kernel(
        _sc_segment_sum_body,
        out_type=[jax.ShapeDtypeStruct((NPAD, CW), jnp.float32)] * NCHUNK,
        mesh=mesh,
        scratch_types=[
            pltpu.VMEM_SHARED((NPAD, CW), jnp.float32),
            pltpu.VMEM((WIN, CW), jnp.float32),
            pltpu.VMEM((WIN, CW), jnp.float32),
            pltpu.VMEM((2, 2, WIN), jnp.int32),
            pltpu.SemaphoreType.DMA,
            pltpu.SemaphoreType.DMA,
            pltpu.SemaphoreType.DMA,
        ],
        compiler_params=pltpu.CompilerParams(use_tc_tiling_on_sc=False),
    )
    return f(*tables, edges3)


def _tc_body(x_ref, a0, a1, a2, a3, wrel_ref, wroot_ref, brel_ref,
             gnw, gnb, gns, bng, bnb, linwt, linb, out_ref,
             ssum, ssq, smax, smin):
    i = pl.program_id(0)

    h = jnp.dot(x_ref[...], wroot_ref[...], preferred_element_type=jnp.float32)
    for q, aq in enumerate((a0, a1, a2, a3)):
        h += jnp.dot(aq[...], wrel_ref[q * CW:(q + 1) * CW, :],
                     preferred_element_type=jnp.float32)
    h += brel_ref[...]

    @pl.when(i == 0)
    def _():
        ssum[...] = jnp.zeros_like(ssum)
        ssq[...] = jnp.zeros_like(ssq)
        smax[...] = jnp.full_like(smax, -jnp.inf)
        smin[...] = jnp.full_like(smin, jnp.inf)

    ssum[...] += jnp.sum(h, axis=0, keepdims=True)
    ssq[...] += jnp.sum(h * h, axis=0, keepdims=True)
    smax[...] = jnp.maximum(smax[...], jnp.max(h, axis=0, keepdims=True))
    smin[...] = jnp.minimum(smin[...], jnp.min(h, axis=0, keepdims=True))

    @pl.when(i == NBLK - 1)
    def _():
        inv_n = 1.0 / N
        mean = ssum[...] * inv_n
        ex2 = ssq[...] * inv_n
        sm = gns[...] * mean
        gnvar = ex2 - 2.0 * sm * mean + sm * sm
        a1v = gnw[...] * lax.rsqrt(gnvar + 1e-5)
        b1v = gnb[...] - a1v * sm
        tvar = ex2 - mean * mean
        m1 = a1v * mean + b1v
        v1 = a1v * a1v * tvar
        a2v = bng[...] * lax.rsqrt(v1 + 1e-5)
        b2v = bnb[...] - a2v * m1
        A = a1v * a2v
        B = a2v * b1v + b2v
        r = jnp.where(A >= 0.0, A * smax[...], A * smin[...]) + B
        out_ref[...] = jnp.dot(r, linwt[...],
                               preferred_element_type=jnp.float32) + linb[...]


def _tc_encode(cell_x, aggs, wrel_t_pad, wroot_t, brel, gnw, gnb, gns,
               bng, bnb, linwt, linb):
    row_spec = pl.BlockSpec((BLK, D), lambda i: (i, 0))
    agg_spec = pl.BlockSpec((BLK, CW), lambda i: (i, 0))
    def whole(shape):
        return pl.BlockSpec(shape, lambda i: tuple(0 for _ in shape))
    return pl.pallas_call(
        _tc_body,
        grid=(NBLK,),
        in_specs=[
            row_spec, agg_spec, agg_spec, agg_spec, agg_spec,
            whole((DPAD, D)), whole((D, D)), whole((1, D)),
            whole((1, D)), whole((1, D)), whole((1, D)),
            whole((1, D)), whole((1, D)),
            whole((D, DOUT)), whole((1, DOUT)),
        ],
        out_specs=whole((1, DOUT)),
        out_shape=jax.ShapeDtypeStruct((1, DOUT), jnp.float32),
        scratch_shapes=[
            pltpu.VMEM((1, D), jnp.float32),
            pltpu.VMEM((1, D), jnp.float32),
            pltpu.VMEM((1, D), jnp.float32),
            pltpu.VMEM((1, D), jnp.float32),
        ],
    )(cell_x, *aggs, wrel_t_pad, wroot_t, brel, gnw, gnb, gns, bng, bnb,
      linwt, linb)


def kernel(cell_x, cell_edge_index, tissue_x, tissue_edge_index,
           assignment_mat, W_rel, b_rel, W_root, gn_weight, gn_bias,
           gn_mean_scale, bn_gamma, bn_beta, lin_W, lin_b):
    del tissue_x, tissue_edge_index, assignment_mat

    src = cell_edge_index[0].astype(jnp.int32)
    dst = cell_edge_index[1].astype(jnp.int32)

    npad_e = EPT_PAD - EPT
    tile_ids = jnp.arange(NS, dtype=jnp.int32)[:, None]
    j = jnp.arange(npad_e, dtype=jnp.int32)[None, :]
    pad_src = (tile_ids * 997 + j * 41) % N
    pad_dst = N + (j + tile_ids * 13) % (NPAD - N)
    src3 = jnp.concatenate([src.reshape(NS, EPT), pad_src], axis=1)
    dst3 = jnp.concatenate([dst.reshape(NS, EPT), pad_dst], axis=1)
    src3 = src3.reshape(NS, NWIN, WIN)
    dst3 = dst3.reshape(NS, NWIN, WIN)
    edges3 = jnp.stack([src3, dst3], axis=2)

    cxp = jnp.pad(cell_x, ((0, 0), (0, DPAD - D)))
    tables = [cxp[:, q * CW:(q + 1) * CW] for q in range(NCHUNK)]

    aggs = _sc_segment_sum(tables, edges3)

    wrel_t_pad = jnp.pad(W_rel.T, ((0, DPAD - D), (0, 0)))
    out = _tc_encode(
        cell_x, aggs, wrel_t_pad, W_root.T, b_rel.reshape(1, D),
        gn_weight.reshape(1, D), gn_bias.reshape(1, D),
        gn_mean_scale.reshape(1, D), bn_gamma.reshape(1, D),
        bn_beta.reshape(1, D), lin_W.T, lin_b.reshape(1, DOUT))
    return out[:, None, :]


# speedup vs baseline: 4.2574x; 1.0153x over previous
import functools

import jax
import jax.numpy as jnp
from jax import lax
from jax.experimental import pallas as pl
from jax.experimental.pallas import tpu as pltpu
from jax.experimental.pallas import tpu_sc as plsc

N = 10000
E = 160000
D = 514
DOUT = 256
NPAD = 10240
CW = 128
NCHUNK = 5
DPAD = NCHUNK * CW

NC, NS = 2, 16
EPT = E // NS
WIN = 128
NWIN = 80
EPT_PAD = NWIN * WIN
ROWS_PER_TILE = NPAD // NS

NBLK = 25
BLK = N // NBLK


def _sc_segment_sum_body(tab, edges4, o0, o1, o2, o3, o4a, o4b,
                         spm, rows0, rows1, exr, gsem0, gsem1, isem):
    c = lax.axis_index("c")
    s = lax.axis_index("s")

    def load_idx(w, slot):
        pltpu.async_copy(edges4.at[s, w], exr.at[slot], isem)

    def wait_idx():
        pltpu.make_async_copy(edges4.at[s, 0], exr.at[0], isem).wait()

    def do_chunk(q, out, w_lo, w_hi):
        nwin = w_hi - w_lo
        def _zrow(r, _):
            def _zcol(k, _):
                rows0[r, pl.ds(k * 16, 16)] = jnp.zeros((16,), jnp.float32)
                return 0
            return lax.fori_loop(0, CW // 16, _zcol, 0)
        lax.fori_loop(0, WIN, _zrow, 0)
        for k in range(ROWS_PER_TILE // WIN):
            pltpu.sync_copy(rows0, spm.at[pl.ds(s * ROWS_PER_TILE + k * WIN, WIN)])
        plsc.subcore_barrier()

        def start(rbuf, slot, sem):
            pltpu.async_copy(tab.at[exr.at[slot, q]], rbuf, sem)

        def wait(rbuf, sem):
            pltpu.make_async_copy(tab.at[exr.at[0, q]], rbuf, sem).wait()

        def scatter(rbuf, slot):
            pltpu.sync_copy(rbuf, spm.at[exr.at[slot, NCHUNK]], add=True)

        pltpu.sync_copy(edges4.at[s, w_lo], exr.at[0])
        start(rows0, 0, gsem0)
        load_idx(w_lo + 1, 1)

        def gbody(g, _):
            b = w_lo + 2 * g + 2
            wait_idx()
            start(rows1, 1, gsem1)
            wait(rows0, gsem0)
            scatter(rows0, 0)
            load_idx(b, 0)
            wait_idx()
            start(rows0, 0, gsem0)
            wait(rows1, gsem1)
            scatter(rows1, 1)
            load_idx(b + 1, 1)
            return 0
        lax.fori_loop(0, nwin // 2 - 1, gbody, 0)
        wait_idx()
        start(rows1, 1, gsem1)
        wait(rows0, gsem0)
        scatter(rows0, 0)
        wait(rows1, gsem1)
        scatter(rows1, 1)

        plsc.subcore_barrier()
        pltpu.sync_copy(spm.at[pl.ds(s * ROWS_PER_TILE, ROWS_PER_TILE)],
                        out.at[pl.ds(s * ROWS_PER_TILE, ROWS_PER_TILE)])
        plsc.subcore_barrier()

    @pl.when(c == 0)
    def _():
        do_chunk(0, o0, 0, NWIN)
        do_chunk(1, o1, 0, NWIN)
        do_chunk(4, o4a, 0, NWIN // 2)

    @pl.when(c == 1)
    def _():
        do_chunk(2, o2, 0, NWIN)
        do_chunk(3, o3, 0, NWIN)
        do_chunk(4, o4b, NWIN // 2, NWIN)


def _sc_segment_sum(tab, edges4):
    mesh = plsc.VectorSubcoreMesh(core_axis_name="c", subcore_axis_name="s",
                                  num_cores=NC, num_subcores=NS)
    f = pl.kernel(
        _sc_segment_sum_body,
        out_type=[jax.ShapeDtypeStruct((NPAD, CW), jnp.float32)] * (NCHUNK + 1),
        mesh=mesh,
        scratch_types=[
            pltpu.VMEM_SHARED((NPAD, CW), jnp.float32),
            pltpu.VMEM((WIN, CW), jnp.float32),
            pltpu.VMEM((WIN, CW), jnp.float32),
            pltpu.VMEM((2, NCHUNK + 1, WIN), jnp.int32),
            pltpu.SemaphoreType.DMA,
            pltpu.SemaphoreType.DMA,
            pltpu.SemaphoreType.DMA,
        ],
    )
    return f(tab, edges4)


def _tc_body(x_ref, a0, a1, a2, a3, a4a, a4b, wrel_ref, wroot_ref, brel_ref,
             gnw, gnb, gns, bng, bnb, linwt, linb, out_ref,
             ssum, ssq, smax, smin):
    i = pl.program_id(0)

    h = jnp.dot(x_ref[...], wroot_ref[...], preferred_element_type=jnp.float32)
    for q, aq in enumerate((a0, a1, a2, a3, a4a, a4b)):
        qq = min(q, 4)
        h += jnp.dot(aq[...], wrel_ref[qq * CW:(qq + 1) * CW, :],
                     preferred_element_type=jnp.float32)
    h += brel_ref[...]

    @pl.when(i == 0)
    def _():
        ssum[...] = jnp.zeros_like(ssum)
        ssq[...] = jnp.zeros_like(ssq)
        smax[...] = jnp.full_like(smax, -jnp.inf)
        smin[...] = jnp.full_like(smin, jnp.inf)

    ssum[...] += jnp.sum(h, axis=0, keepdims=True)
    ssq[...] += jnp.sum(h * h, axis=0, keepdims=True)
    smax[...] = jnp.maximum(smax[...], jnp.max(h, axis=0, keepdims=True))
    smin[...] = jnp.minimum(smin[...], jnp.min(h, axis=0, keepdims=True))

    @pl.when(i == NBLK - 1)
    def _():
        inv_n = 1.0 / N
        mean = ssum[...] * inv_n
        ex2 = ssq[...] * inv_n
        sm = gns[...] * mean
        gnvar = ex2 - 2.0 * sm * mean + sm * sm
        a1v = gnw[...] * lax.rsqrt(gnvar + 1e-5)
        b1v = gnb[...] - a1v * sm
        tvar = ex2 - mean * mean
        m1 = a1v * mean + b1v
        v1 = a1v * a1v * tvar
        a2v = bng[...] * lax.rsqrt(v1 + 1e-5)
        b2v = bnb[...] - a2v * m1
        A = a1v * a2v
        B = a2v * b1v + b2v
        r = jnp.where(A >= 0.0, A * smax[...], A * smin[...]) + B
        out_ref[...] = jnp.dot(r, linwt[...],
                               preferred_element_type=jnp.float32) + linb[...]


def _tc_encode(cell_x, aggs, wrel_t_pad, wroot_t, brel, gnw, gnb, gns,
               bng, bnb, linwt, linb):
    row_spec = pl.BlockSpec((BLK, D), lambda i: (i, 0))
    agg_spec = pl.BlockSpec((BLK, CW), lambda i: (i, 0))
    def whole(shape):
        return pl.BlockSpec(shape, lambda i: tuple(0 for _ in shape))
    return pl.pallas_call(
        _tc_body,
        grid=(NBLK,),
        in_specs=[
            row_spec, agg_spec, agg_spec, agg_spec, agg_spec, agg_spec,
            agg_spec,
            whole((DPAD, D)), whole((D, D)), whole((1, D)),
            whole((1, D)), whole((1, D)), whole((1, D)),
            whole((1, D)), whole((1, D)),
            whole((D, DOUT)), whole((1, DOUT)),
        ],
        out_specs=whole((1, DOUT)),
        out_shape=jax.ShapeDtypeStruct((1, DOUT), jnp.float32),
        scratch_shapes=[
            pltpu.VMEM((1, D), jnp.float32),
            pltpu.VMEM((1, D), jnp.float32),
            pltpu.VMEM((1, D), jnp.float32),
            pltpu.VMEM((1, D), jnp.float32),
        ],
    )(cell_x, *aggs, wrel_t_pad, wroot_t, brel, gnw, gnb, gns, bng, bnb,
      linwt, linb)


def kernel(cell_x, cell_edge_index, tissue_x, tissue_edge_index,
           assignment_mat, W_rel, b_rel, W_root, gn_weight, gn_bias,
           gn_mean_scale, bn_gamma, bn_beta, lin_W, lin_b):
    del tissue_x, tissue_edge_index, assignment_mat

    src = cell_edge_index[0].astype(jnp.int32)
    dst = cell_edge_index[1].astype(jnp.int32)

    npad_e = EPT_PAD - EPT
    tile_ids = jnp.arange(NS, dtype=jnp.int32)[:, None]
    j = jnp.arange(npad_e, dtype=jnp.int32)[None, :]
    pad_src = (tile_ids * 997 + j * 41) % N
    pad_dst = N + (j + tile_ids * 13) % (NPAD - N)
    src2 = jnp.concatenate([src.reshape(NS, EPT), pad_src], axis=1)
    dst2 = jnp.concatenate([dst.reshape(NS, EPT), pad_dst], axis=1)
    src2 = src2.reshape(NS, NWIN, 1, WIN) * NCHUNK
    dst2 = dst2.reshape(NS, NWIN, 1, WIN)
    qoff = jnp.arange(NCHUNK, dtype=jnp.int32).reshape(1, 1, NCHUNK, 1)
    edges4 = jnp.concatenate([src2 + qoff, dst2], axis=2)

    tab = jnp.pad(cell_x, ((0, 0), (0, DPAD - D))).reshape(N * NCHUNK, CW)

    aggs = _sc_segment_sum(tab, edges4)

    wrel_t_pad = jnp.pad(W_rel.T, ((0, DPAD - D), (0, 0)))
    out = _tc_encode(
        cell_x, aggs, wrel_t_pad, W_root.T, b_rel.reshape(1, D),
        gn_weight.reshape(1, D), gn_bias.reshape(1, D),
        gn_mean_scale.reshape(1, D), bn_gamma.reshape(1, D),
        bn_beta.reshape(1, D), lin_W.T, lin_b.reshape(1, DOUT))
    return out[:, None, :]


# speedup vs baseline: 4.4549x; 1.0464x over previous
import functools

import jax
import jax.numpy as jnp
from jax import lax
from jax.experimental import pallas as pl
from jax.experimental.pallas import tpu as pltpu
from jax.experimental.pallas import tpu_sc as plsc

N = 10000
E = 160000
D = 514
DOUT = 256
NPAD = 10240
CW = 128
NCHUNK = 5
DPAD = NCHUNK * CW

NC, NS = 2, 16
EPT = E // NS
WIN = 128
NWIN = 80
EPT_PAD = NWIN * WIN
ROWS_PER_TILE = NPAD // NS

NBLK = 25
BLK = N // NBLK


def _sc_segment_sum_body(tab, edges4, o0, o1, o2, o3, o4a, o4b,
                         spm, rows0, rows1, exr, gsem0, gsem1, isem):
    c = lax.axis_index("c")
    s = lax.axis_index("s")

    def load_idx(w, slot):
        pltpu.async_copy(edges4.at[s, w], exr.at[slot], isem)

    def wait_idx():
        pltpu.make_async_copy(edges4.at[s, 0], exr.at[0], isem).wait()

    def do_chunk(q, out, w_lo, w_hi):
        nwin = w_hi - w_lo
        def _zrow(r, _):
            def _zcol(k, _):
                rows0[r, pl.ds(k * 16, 16)] = jnp.zeros((16,), jnp.float32)
                return 0
            return lax.fori_loop(0, CW // 16, _zcol, 0)
        lax.fori_loop(0, WIN, _zrow, 0)
        for k in range(ROWS_PER_TILE // WIN):
            pltpu.sync_copy(rows0, spm.at[pl.ds(s * ROWS_PER_TILE + k * WIN, WIN)])
        plsc.subcore_barrier()

        def start(rbuf, slot, sem):
            pltpu.async_copy(tab.at[exr.at[slot, q]], rbuf, sem)

        def wait(rbuf, sem):
            pltpu.make_async_copy(tab.at[exr.at[0, q]], rbuf, sem).wait()

        def scatter(rbuf, slot):
            pltpu.sync_copy(rbuf, spm.at[exr.at[slot, NCHUNK]], add=True)

        pltpu.sync_copy(edges4.at[s, w_lo], exr.at[0])
        start(rows0, 0, gsem0)
        load_idx(w_lo + 1, 1)

        def gbody(g, _):
            b = w_lo + 2 * g + 2
            wait_idx()
            start(rows1, 1, gsem1)
            wait(rows0, gsem0)
            scatter(rows0, 0)
            load_idx(b, 0)
            wait_idx()
            start(rows0, 0, gsem0)
            wait(rows1, gsem1)
            scatter(rows1, 1)
            load_idx(b + 1, 1)
            return 0
        lax.fori_loop(0, nwin // 2 - 1, gbody, 0)
        wait_idx()
        start(rows1, 1, gsem1)
        wait(rows0, gsem0)
        scatter(rows0, 0)
        wait(rows1, gsem1)
        scatter(rows1, 1)

        plsc.subcore_barrier()
        pltpu.sync_copy(spm.at[pl.ds(s * ROWS_PER_TILE, ROWS_PER_TILE)],
                        out.at[pl.ds(s * ROWS_PER_TILE, ROWS_PER_TILE)])
        plsc.subcore_barrier()

    @pl.when(c == 0)
    def _():
        do_chunk(0, o0, 0, NWIN)
        do_chunk(1, o1, 0, NWIN)
        do_chunk(4, o4a, 0, NWIN // 2)

    @pl.when(c == 1)
    def _():
        do_chunk(2, o2, 0, NWIN)
        do_chunk(3, o3, 0, NWIN)
        do_chunk(4, o4b, NWIN // 2, NWIN)


def _sc_segment_sum(tab, edges4):
    mesh = plsc.VectorSubcoreMesh(core_axis_name="c", subcore_axis_name="s",
                                  num_cores=NC, num_subcores=NS)
    f = pl.kernel(
        _sc_segment_sum_body,
        out_type=[jax.ShapeDtypeStruct((NPAD, CW), jnp.float32)] * (NCHUNK + 1),
        mesh=mesh,
        scratch_types=[
            pltpu.VMEM_SHARED((NPAD, CW), jnp.float32),
            pltpu.VMEM((WIN, CW), jnp.float32),
            pltpu.VMEM((WIN, CW), jnp.float32),
            pltpu.VMEM((2, NCHUNK + 1, WIN), jnp.int32),
            pltpu.SemaphoreType.DMA,
            pltpu.SemaphoreType.DMA,
            pltpu.SemaphoreType.DMA,
        ],
    )
    return f(tab, edges4)


def _dot_t(a, w):
    return lax.dot_general(a, w, (((1,), (1,)), ((), ())),
                           preferred_element_type=jnp.float32)


def _tc_body(x_ref, a0, a1, a2, a3, a4a, a4b, wrel_ref, wroot_ref, brel_ref,
             gnw, gnb, gns, bng, bnb, linw, linb, out_ref,
             ssum, ssq, smax, smin):
    i = pl.program_id(0)

    h = _dot_t(x_ref[...], wroot_ref[...])
    for q, aq in enumerate((a0, a1, a2, a3)):
        h += _dot_t(aq[...], wrel_ref[:, q * CW:(q + 1) * CW])
    a4 = a4a[:, 0:2] + a4b[:, 0:2]
    h += _dot_t(a4, wrel_ref[:, 4 * CW:D])
    h += brel_ref[...]

    @pl.when(i == 0)
    def _():
        ssum[...] = jnp.zeros_like(ssum)
        ssq[...] = jnp.zeros_like(ssq)
        smax[...] = jnp.full_like(smax, -jnp.inf)
        smin[...] = jnp.full_like(smin, jnp.inf)

    ssum[...] += jnp.sum(h, axis=0, keepdims=True)
    ssq[...] += jnp.sum(h * h, axis=0, keepdims=True)
    smax[...] = jnp.maximum(smax[...], jnp.max(h, axis=0, keepdims=True))
    smin[...] = jnp.minimum(smin[...], jnp.min(h, axis=0, keepdims=True))

    @pl.when(i == NBLK - 1)
    def _():
        inv_n = 1.0 / N
        mean = ssum[...] * inv_n
        ex2 = ssq[...] * inv_n
        sm = gns[...] * mean
        gnvar = ex2 - 2.0 * sm * mean + sm * sm
        a1v = gnw[...] * lax.rsqrt(gnvar + 1e-5)
        b1v = gnb[...] - a1v * sm
        tvar = ex2 - mean * mean
        m1 = a1v * mean + b1v
        v1 = a1v * a1v * tvar
        a2v = bng[...] * lax.rsqrt(v1 + 1e-5)
        b2v = bnb[...] - a2v * m1
        A = a1v * a2v
        B = a2v * b1v + b2v
        r = jnp.where(A >= 0.0, A * smax[...], A * smin[...]) + B
        out_ref[...] = _dot_t(r, linw[...]) + linb[...]


def _tc_encode(cell_x, aggs, wrel, wroot, brel, gnw, gnb, gns,
               bng, bnb, linw, linb):
    row_spec = pl.BlockSpec((BLK, D), lambda i: (i, 0))
    agg_spec = pl.BlockSpec((BLK, CW), lambda i: (i, 0))
    def whole(shape):
        return pl.BlockSpec(shape, lambda i: tuple(0 for _ in shape))
    return pl.pallas_call(
        _tc_body,
        grid=(NBLK,),
        in_specs=[
            row_spec, agg_spec, agg_spec, agg_spec, agg_spec, agg_spec,
            agg_spec,
            whole((D, D)), whole((D, D)), whole((1, D)),
            whole((1, D)), whole((1, D)), whole((1, D)),
            whole((1, D)), whole((1, D)),
            whole((DOUT, D)), whole((1, DOUT)),
        ],
        out_specs=whole((1, DOUT)),
        out_shape=jax.ShapeDtypeStruct((1, DOUT), jnp.float32),
        scratch_shapes=[
            pltpu.VMEM((1, D), jnp.float32),
            pltpu.VMEM((1, D), jnp.float32),
            pltpu.VMEM((1, D), jnp.float32),
            pltpu.VMEM((1, D), jnp.float32),
        ],
    )(cell_x, *aggs, wrel, wroot, brel, gnw, gnb, gns, bng, bnb,
      linw, linb)


def kernel(cell_x, cell_edge_index, tissue_x, tissue_edge_index,
           assignment_mat, W_rel, b_rel, W_root, gn_weight, gn_bias,
           gn_mean_scale, bn_gamma, bn_beta, lin_W, lin_b):
    del tissue_x, tissue_edge_index, assignment_mat

    src = cell_edge_index[0].astype(jnp.int32)
    dst = cell_edge_index[1].astype(jnp.int32)

    npad_e = EPT_PAD - EPT
    tile_ids = jnp.arange(NS, dtype=jnp.int32)[:, None]
    j = jnp.arange(npad_e, dtype=jnp.int32)[None, :]
    pad_src = (tile_ids * 997 + j * 41) % N
    pad_dst = N + (j + tile_ids * 13) % (NPAD - N)
    src2 = jnp.concatenate([src.reshape(NS, EPT), pad_src], axis=1)
    dst2 = jnp.concatenate([dst.reshape(NS, EPT), pad_dst], axis=1)
    src2 = src2.reshape(NS, NWIN, 1, WIN)
    dst2 = dst2.reshape(NS, NWIN, 1, WIN)
    qoff = (jnp.arange(NCHUNK, dtype=jnp.int32) * N).reshape(1, 1, NCHUNK, 1)
    edges4 = jnp.concatenate([src2 + qoff, dst2], axis=2)

    tab = jnp.concatenate(
        [cell_x[:, q * CW:(q + 1) * CW] for q in range(NCHUNK - 1)]
        + [jnp.pad(cell_x[:, (NCHUNK - 1) * CW:], ((0, 0), (0, DPAD - D)))],
        axis=0)

    aggs = _sc_segment_sum(tab, edges4)

    out = _tc_encode(
        cell_x, aggs, W_rel, W_root, b_rel.reshape(1, D),
        gn_weight.reshape(1, D), gn_bias.reshape(1, D),
        gn_mean_scale.reshape(1, D), bn_gamma.reshape(1, D),
        bn_beta.reshape(1, D), lin_W, lin_b.reshape(1, DOUT))
    return out[:, None, :]


# speedup vs baseline: 4.4567x; 1.0004x over previous
import functools

import jax
import jax.numpy as jnp
from jax import lax
from jax.experimental import pallas as pl
from jax.experimental.pallas import tpu as pltpu
from jax.experimental.pallas import tpu_sc as plsc

N = 10000
E = 160000
D = 514
DOUT = 256
NPAD = 10240
CW = 128
NCHUNK = 5
DPAD = NCHUNK * CW

NC, NS = 2, 16
EPT = E // NS
WIN = 128
NWIN = 80
EPT_PAD = NWIN * WIN
ROWS_PER_TILE = NPAD // NS

NBLK = 25
BLK = N // NBLK


def _sc_segment_sum_body(tab, edges4, o0, o1, o2, o3, o4a, o4b,
                         spm, rows0, rows1, exr, gsem0, gsem1, isem):
    c = lax.axis_index("c")
    s = lax.axis_index("s")

    def load_idx(w, slot):
        pltpu.async_copy(edges4.at[s, w], exr.at[slot], isem)

    def wait_idx():
        pltpu.make_async_copy(edges4.at[s, 0], exr.at[0], isem).wait()

    def do_chunk(q, out, w_lo, w_hi):
        nwin = w_hi - w_lo
        def _zrow(r, _):
            def _zcol(k, _):
                rows0[r, pl.ds(k * 16, 16)] = jnp.zeros((16,), jnp.float32)
                return 0
            return lax.fori_loop(0, CW // 16, _zcol, 0)
        lax.fori_loop(0, WIN, _zrow, 0)
        for k in range(ROWS_PER_TILE // WIN):
            pltpu.sync_copy(rows0, spm.at[pl.ds(s * ROWS_PER_TILE + k * WIN, WIN)])
        plsc.subcore_barrier()

        def start(rbuf, slot, sem):
            pltpu.async_copy(tab.at[exr.at[slot, q]], rbuf, sem)

        def wait(rbuf, sem):
            pltpu.make_async_copy(tab.at[exr.at[0, q]], rbuf, sem).wait()

        def scatter(rbuf, slot):
            pltpu.sync_copy(rbuf, spm.at[exr.at[slot, NCHUNK]], add=True)

        pltpu.sync_copy(edges4.at[s, w_lo], exr.at[0])
        start(rows0, 0, gsem0)
        load_idx(w_lo + 1, 1)

        def gbody(g, _):
            b = w_lo + 2 * g + 2
            wait_idx()
            start(rows1, 1, gsem1)
            wait(rows0, gsem0)
            scatter(rows0, 0)
            load_idx(b, 0)
            wait_idx()
            start(rows0, 0, gsem0)
            wait(rows1, gsem1)
            scatter(rows1, 1)
            load_idx(b + 1, 1)
            return 0
        lax.fori_loop(0, nwin // 2 - 1, gbody, 0)
        wait_idx()
        start(rows1, 1, gsem1)
        wait(rows0, gsem0)
        scatter(rows0, 0)
        wait(rows1, gsem1)
        scatter(rows1, 1)

        plsc.subcore_barrier()
        pltpu.sync_copy(spm.at[pl.ds(s * ROWS_PER_TILE, ROWS_PER_TILE)],
                        out.at[pl.ds(s * ROWS_PER_TILE, ROWS_PER_TILE)])
        plsc.subcore_barrier()

    @pl.when(c == 0)
    def _():
        do_chunk(0, o0, 0, NWIN)
        do_chunk(1, o1, 0, NWIN)
        do_chunk(4, o4a, 0, NWIN // 2)

    @pl.when(c == 1)
    def _():
        do_chunk(2, o2, 0, NWIN)
        do_chunk(3, o3, 0, NWIN)
        do_chunk(4, o4b, NWIN // 2, NWIN)


def _sc_segment_sum(tab, edges4):
    mesh = plsc.VectorSubcoreMesh(core_axis_name="c", subcore_axis_name="s",
                                  num_cores=NC, num_subcores=NS)
    f = pl.kernel(
        _sc_segment_sum_body,
        out_type=[jax.ShapeDtypeStruct((NPAD, CW), jnp.float32)] * (NCHUNK + 1),
        mesh=mesh,
        scratch_types=[
            pltpu.VMEM_SHARED((NPAD, CW), jnp.float32),
            pltpu.VMEM((WIN, CW), jnp.float32),
            pltpu.VMEM((WIN, CW), jnp.float32),
            pltpu.VMEM((2, NCHUNK + 1, WIN), jnp.int32),
            pltpu.SemaphoreType.DMA,
            pltpu.SemaphoreType.DMA,
            pltpu.SemaphoreType.DMA,
        ],
    )
    return f(tab, edges4)


def _dot_t(a, w):
    return lax.dot_general(a, w, (((1,), (1,)), ((), ())),
                           preferred_element_type=jnp.float32)


def _xroot_body(x_ref, wroot_ref, brel_ref, out_ref):
    out_ref[...] = _dot_t(x_ref[...], wroot_ref[...]) + brel_ref[...]


def _xroot(cell_x, wroot, brel):
    blk = 2000
    return pl.pallas_call(
        _xroot_body,
        grid=(N // blk,),
        in_specs=[
            pl.BlockSpec((blk, D), lambda i: (i, 0)),
            pl.BlockSpec((D, D), lambda i: (0, 0)),
            pl.BlockSpec((1, D), lambda i: (0, 0)),
        ],
        out_specs=pl.BlockSpec((blk, D), lambda i: (i, 0)),
        out_shape=jax.ShapeDtypeStruct((N, D), jnp.float32),
    )(cell_x, wroot, brel)


def _tc_body(xr_ref, a0, a1, a2, a3, a4a, a4b, wrel_ref,
             gnw, gnb, gns, bng, bnb, linw, linb, out_ref,
             ssum, ssq, smax, smin):
    i = pl.program_id(0)

    h = xr_ref[...]
    for q, aq in enumerate((a0, a1, a2, a3)):
        h += _dot_t(aq[...], wrel_ref[:, q * CW:(q + 1) * CW])
    a4 = a4a[:, 0:2] + a4b[:, 0:2]
    h += _dot_t(a4, wrel_ref[:, 4 * CW:D])

    @pl.when(i == 0)
    def _():
        ssum[...] = jnp.zeros_like(ssum)
        ssq[...] = jnp.zeros_like(ssq)
        smax[...] = jnp.full_like(smax, -jnp.inf)
        smin[...] = jnp.full_like(smin, jnp.inf)

    ssum[...] += jnp.sum(h, axis=0, keepdims=True)
    ssq[...] += jnp.sum(h * h, axis=0, keepdims=True)
    smax[...] = jnp.maximum(smax[...], jnp.max(h, axis=0, keepdims=True))
    smin[...] = jnp.minimum(smin[...], jnp.min(h, axis=0, keepdims=True))

    @pl.when(i == NBLK - 1)
    def _():
        inv_n = 1.0 / N
        mean = ssum[...] * inv_n
        ex2 = ssq[...] * inv_n
        sm = gns[...] * mean
        gnvar = ex2 - 2.0 * sm * mean + sm * sm
        a1v = gnw[...] * lax.rsqrt(gnvar + 1e-5)
        b1v = gnb[...] - a1v * sm
        tvar = ex2 - mean * mean
        m1 = a1v * mean + b1v
        v1 = a1v * a1v * tvar
        a2v = bng[...] * lax.rsqrt(v1 + 1e-5)
        b2v = bnb[...] - a2v * m1
        A = a1v * a2v
        B = a2v * b1v + b2v
        r = jnp.where(A >= 0.0, A * smax[...], A * smin[...]) + B
        out_ref[...] = _dot_t(r, linw[...]) + linb[...]


def _tc_encode(xr, aggs, wrel, gnw, gnb, gns, bng, bnb, linw, linb):
    row_spec = pl.BlockSpec((BLK, D), lambda i: (i, 0))
    agg_spec = pl.BlockSpec((BLK, CW), lambda i: (i, 0))
    def whole(shape):
        return pl.BlockSpec(shape, lambda i: tuple(0 for _ in shape))
    return pl.pallas_call(
        _tc_body,
        grid=(NBLK,),
        in_specs=[
            row_spec, agg_spec, agg_spec, agg_spec, agg_spec, agg_spec,
            agg_spec,
            whole((D, D)),
            whole((1, D)), whole((1, D)), whole((1, D)),
            whole((1, D)), whole((1, D)),
            whole((DOUT, D)), whole((1, DOUT)),
        ],
        out_specs=whole((1, DOUT)),
        out_shape=jax.ShapeDtypeStruct((1, DOUT), jnp.float32),
        scratch_shapes=[
            pltpu.VMEM((1, D), jnp.float32),
            pltpu.VMEM((1, D), jnp.float32),
            pltpu.VMEM((1, D), jnp.float32),
            pltpu.VMEM((1, D), jnp.float32),
        ],
    )(xr, *aggs, wrel, gnw, gnb, gns, bng, bnb, linw, linb)


def kernel(cell_x, cell_edge_index, tissue_x, tissue_edge_index,
           assignment_mat, W_rel, b_rel, W_root, gn_weight, gn_bias,
           gn_mean_scale, bn_gamma, bn_beta, lin_W, lin_b):
    del tissue_x, tissue_edge_index, assignment_mat

    src = cell_edge_index[0].astype(jnp.int32)
    dst = cell_edge_index[1].astype(jnp.int32)

    npad_e = EPT_PAD - EPT
    tile_ids = jnp.arange(NS, dtype=jnp.int32)[:, None]
    j = jnp.arange(npad_e, dtype=jnp.int32)[None, :]
    pad_src = (tile_ids * 997 + j * 41) % N
    pad_dst = N + (j + tile_ids * 13) % (NPAD - N)
    src2 = jnp.concatenate([src.reshape(NS, EPT), pad_src], axis=1)
    dst2 = jnp.concatenate([dst.reshape(NS, EPT), pad_dst], axis=1)
    src2 = src2.reshape(NS, NWIN, 1, WIN)
    dst2 = dst2.reshape(NS, NWIN, 1, WIN)
    qoff = (jnp.arange(NCHUNK, dtype=jnp.int32) * N).reshape(1, 1, NCHUNK, 1)
    edges4 = jnp.concatenate([src2 + qoff, dst2], axis=2)

    tab = jnp.concatenate(
        [cell_x[:, q * CW:(q + 1) * CW] for q in range(NCHUNK - 1)]
        + [jnp.pad(cell_x[:, (NCHUNK - 1) * CW:], ((0, 0), (0, DPAD - D)))],
        axis=0)

    aggs = _sc_segment_sum(tab, edges4)
    xr = _xroot(cell_x, W_root, b_rel.reshape(1, D))

    out = _tc_encode(
        xr, aggs, W_rel,
        gn_weight.reshape(1, D), gn_bias.reshape(1, D),
        gn_mean_scale.reshape(1, D), bn_gamma.reshape(1, D),
        bn_beta.reshape(1, D), lin_W, lin_b.reshape(1, DOUT))
    return out[:, None, :]


# speedup vs baseline: 4.5043x; 1.0107x over previous
import functools

import jax
import jax.numpy as jnp
from jax import lax
from jax.experimental import pallas as pl
from jax.experimental.pallas import tpu as pltpu
from jax.experimental.pallas import tpu_sc as plsc

N = 10000
E = 160000
D = 514
DOUT = 256
NPAD = 10240
CW = 128
NCHUNK = 5
DPAD = NCHUNK * CW

NC, NS = 2, 16
EPT = E // NS
WIN = 128
NWIN = 80
EPT_PAD = NWIN * WIN
ROWS_PER_TILE = NPAD // NS

NBLK = 25
BLK = N // NBLK


def _sc_segment_sum_body(tab, edges4, o0, o1, o2, o3, o4a, o4b,
                         spm, rows0, rows1, exr, gsem0, gsem1, isem):
    c = lax.axis_index("c")
    s = lax.axis_index("s")

    def load_idx(w, slot):
        pltpu.async_copy(edges4.at[s, w], exr.at[slot], isem)

    def wait_idx():
        pltpu.make_async_copy(edges4.at[s, 0], exr.at[0], isem).wait()

    def do_chunk(q, out, w_lo, w_hi):
        nwin = w_hi - w_lo
        def _zrow(r, _):
            def _zcol(k, _):
                rows0[r, pl.ds(k * 16, 16)] = jnp.zeros((16,), jnp.float32)
                return 0
            return lax.fori_loop(0, CW // 16, _zcol, 0)
        lax.fori_loop(0, WIN, _zrow, 0)
        for k in range(ROWS_PER_TILE // WIN):
            pltpu.sync_copy(rows0, spm.at[pl.ds(s * ROWS_PER_TILE + k * WIN, WIN)])
        plsc.subcore_barrier()

        def start(rbuf, slot, sem):
            pltpu.async_copy(tab.at[exr.at[slot, q]], rbuf, sem)

        def wait(rbuf, sem):
            pltpu.make_async_copy(tab.at[exr.at[0, q]], rbuf, sem).wait()

        def scatter(rbuf, slot):
            pltpu.sync_copy(rbuf, spm.at[exr.at[slot, NCHUNK]], add=True)

        pltpu.sync_copy(edges4.at[s, w_lo], exr.at[0])
        start(rows0, 0, gsem0)
        load_idx(w_lo + 1, 1)

        def gbody(g, _):
            b = w_lo + 2 * g + 2
            wait_idx()
            start(rows1, 1, gsem1)
            wait(rows0, gsem0)
            scatter(rows0, 0)
            load_idx(b, 0)
            wait_idx()
            start(rows0, 0, gsem0)
            wait(rows1, gsem1)
            scatter(rows1, 1)
            load_idx(b + 1, 1)
            return 0
        lax.fori_loop(0, nwin // 2 - 1, gbody, 0)
        wait_idx()
        start(rows1, 1, gsem1)
        wait(rows0, gsem0)
        scatter(rows0, 0)
        wait(rows1, gsem1)
        scatter(rows1, 1)

        plsc.subcore_barrier()
        pltpu.sync_copy(spm.at[pl.ds(s * ROWS_PER_TILE, ROWS_PER_TILE)],
                        out.at[pl.ds(s * ROWS_PER_TILE, ROWS_PER_TILE)])
        plsc.subcore_barrier()

    @pl.when(c == 0)
    def _():
        do_chunk(0, o0, 0, NWIN)
        do_chunk(1, o1, 0, NWIN)
        do_chunk(4, o4a, 0, NWIN // 2)

    @pl.when(c == 1)
    def _():
        do_chunk(2, o2, 0, NWIN)
        do_chunk(3, o3, 0, NWIN)
        do_chunk(4, o4b, NWIN // 2, NWIN)


def _sc_segment_sum(tab, edges4):
    mesh = plsc.VectorSubcoreMesh(core_axis_name="c", subcore_axis_name="s",
                                  num_cores=NC, num_subcores=NS)
    f = pl.kernel(
        _sc_segment_sum_body,
        out_type=[jax.ShapeDtypeStruct((NPAD, CW), jnp.float32)] * (NCHUNK + 1),
        mesh=mesh,
        scratch_types=[
            pltpu.VMEM_SHARED((NPAD, CW), jnp.float32),
            pltpu.VMEM((WIN, CW), jnp.float32),
            pltpu.VMEM((WIN, CW), jnp.float32),
            pltpu.VMEM((2, NCHUNK + 1, WIN), jnp.int32),
            pltpu.SemaphoreType.DMA,
            pltpu.SemaphoreType.DMA,
            pltpu.SemaphoreType.DMA,
        ],
    )
    return f(tab, edges4)


def _dot_t(a, w):
    return lax.dot_general(a, w, (((1,), (1,)), ((), ())),
                           preferred_element_type=jnp.float32)


def _xroot_body(x_ref, wroot_ref, brel_ref, out_ref):
    out_ref[...] = _dot_t(x_ref[...], wroot_ref[...]) + brel_ref[...]


def _xroot(cell_x, wroot, brel):
    blk = 2000
    return pl.pallas_call(
        _xroot_body,
        grid=(N // blk,),
        in_specs=[
            pl.BlockSpec((blk, D), lambda i: (i, 0)),
            pl.BlockSpec((D, D), lambda i: (0, 0)),
            pl.BlockSpec((1, D), lambda i: (0, 0)),
        ],
        out_specs=pl.BlockSpec((blk, D), lambda i: (i, 0)),
        out_shape=jax.ShapeDtypeStruct((N, D), jnp.float32),
    )(cell_x, wroot, brel)


def _tc_body(xr_ref, a0, a1, a2, a3, a4a, a4b, wrel_ref,
             gnw, gnb, gns, bng, bnb, linw, linb, out_ref,
             ssum, ssq, smax, smin):
    i = pl.program_id(0)

    h = xr_ref[...]
    for q, aq in enumerate((a0, a1, a2, a3)):
        h += _dot_t(aq[...], wrel_ref[:, q * CW:(q + 1) * CW])
    a4 = a4a[:, 0:2] + a4b[:, 0:2]
    h += _dot_t(a4, wrel_ref[:, 4 * CW:D])

    @pl.when(i == 0)
    def _():
        ssum[...] = jnp.zeros_like(ssum)
        ssq[...] = jnp.zeros_like(ssq)
        smax[...] = jnp.full_like(smax, -jnp.inf)
        smin[...] = jnp.full_like(smin, jnp.inf)

    ssum[...] += jnp.sum(h, axis=0, keepdims=True)
    ssq[...] += jnp.sum(h * h, axis=0, keepdims=True)
    smax[...] = jnp.maximum(smax[...], jnp.max(h, axis=0, keepdims=True))
    smin[...] = jnp.minimum(smin[...], jnp.min(h, axis=0, keepdims=True))

    @pl.when(i == NBLK - 1)
    def _():
        inv_n = 1.0 / N
        mean = ssum[...] * inv_n
        ex2 = ssq[...] * inv_n
        sm = gns[...] * mean
        gnvar = ex2 - 2.0 * sm * mean + sm * sm
        a1v = gnw[...] * lax.rsqrt(gnvar + 1e-5)
        b1v = gnb[...] - a1v * sm
        tvar = ex2 - mean * mean
        m1 = a1v * mean + b1v
        v1 = a1v * a1v * tvar
        a2v = bng[...] * lax.rsqrt(v1 + 1e-5)
        b2v = bnb[...] - a2v * m1
        A = a1v * a2v
        B = a2v * b1v + b2v
        r = jnp.where(A >= 0.0, A * smax[...], A * smin[...]) + B
        out_ref[...] = _dot_t(r, linw[...]) + linb[...]


def _tc_encode(xr, aggs, wrel, gnw, gnb, gns, bng, bnb, linw, linb):
    row_spec = pl.BlockSpec((BLK, D), lambda i: (i, 0))
    agg_spec = pl.BlockSpec((BLK, CW), lambda i: (i, 0))
    def whole(shape):
        return pl.BlockSpec(shape, lambda i: tuple(0 for _ in shape))
    return pl.pallas_call(
        _tc_body,
        grid=(NBLK,),
        in_specs=[
            row_spec, agg_spec, agg_spec, agg_spec, agg_spec, agg_spec,
            agg_spec,
            whole((D, D)),
            whole((1, D)), whole((1, D)), whole((1, D)),
            whole((1, D)), whole((1, D)),
            whole((DOUT, D)), whole((1, DOUT)),
        ],
        out_specs=whole((1, DOUT)),
        out_shape=jax.ShapeDtypeStruct((1, DOUT), jnp.float32),
        scratch_shapes=[
            pltpu.VMEM((1, D), jnp.float32),
            pltpu.VMEM((1, D), jnp.float32),
            pltpu.VMEM((1, D), jnp.float32),
            pltpu.VMEM((1, D), jnp.float32),
        ],
    )(xr, *aggs, wrel, gnw, gnb, gns, bng, bnb, linw, linb)


def kernel(cell_x, cell_edge_index, tissue_x, tissue_edge_index,
           assignment_mat, W_rel, b_rel, W_root, gn_weight, gn_bias,
           gn_mean_scale, bn_gamma, bn_beta, lin_W, lin_b):
    del tissue_x, tissue_edge_index, assignment_mat

    src = cell_edge_index[0].astype(jnp.int32)
    dst = cell_edge_index[1].astype(jnp.int32)

    npad_e = EPT_PAD - EPT
    tile_ids = jnp.arange(NS, dtype=jnp.int32)[:, None]
    j = jnp.arange(npad_e, dtype=jnp.int32)[None, :]
    pad_src = (tile_ids * 997 + j * 41) % N
    pad_dst = N + (j + tile_ids * 13) % (NPAD - N)
    src2 = jnp.concatenate([src.reshape(NS, EPT), pad_src], axis=1)
    dst2 = jnp.concatenate([dst.reshape(NS, EPT), pad_dst], axis=1)
    src2 = src2.reshape(NS, NWIN, 1, WIN)
    dst2 = dst2.reshape(NS, NWIN, 1, WIN)
    p = jnp.arange(NCHUNK + 1, dtype=jnp.int32).reshape(1, 1, NCHUNK + 1, 1)
    edges4 = jnp.where(p < NCHUNK, src2 + p * N, dst2)

    tab = jnp.concatenate(
        [cell_x[:, q * CW:(q + 1) * CW] for q in range(NCHUNK - 1)]
        + [jnp.pad(cell_x[:, (NCHUNK - 1) * CW:], ((0, 0), (0, DPAD - D)))],
        axis=0)

    aggs = _sc_segment_sum(tab, edges4)
    xr = _xroot(cell_x, W_root, b_rel.reshape(1, D))

    out = _tc_encode(
        xr, aggs, W_rel,
        gn_weight.reshape(1, D), gn_bias.reshape(1, D),
        gn_mean_scale.reshape(1, D), bn_gamma.reshape(1, D),
        bn_beta.reshape(1, D), lin_W, lin_b.reshape(1, DOUT))
    return out[:, None, :]


# speedup vs baseline: 4.5696x; 1.0145x over previous
import functools

import jax
import jax.numpy as jnp
from jax import lax
from jax.experimental import pallas as pl
from jax.experimental.pallas import tpu as pltpu
from jax.experimental.pallas import tpu_sc as plsc

N = 10000
E = 160000
D = 514
DOUT = 256
NPAD = 10240
CW = 128
NCHUNK = 5
DPAD = NCHUNK * CW

NC, NS = 2, 16
EPT = E // NS
WIN = 128
NWIN = 80
EPT_PAD = NWIN * WIN
ROWS_PER_TILE = NPAD // NS

NBLK = 5
BLK = N // NBLK


def _sc_segment_sum_body(tab, edges4, o0, o1, o2, o3, o4a, o4b,
                         spm, rows0, rows1, exr, gsem0, gsem1, isem):
    c = lax.axis_index("c")
    s = lax.axis_index("s")

    def load_idx(w, slot):
        pltpu.async_copy(edges4.at[s, w], exr.at[slot], isem)

    def wait_idx():
        pltpu.make_async_copy(edges4.at[s, 0], exr.at[0], isem).wait()

    def do_chunk(q, out, w_lo, w_hi):
        nwin = w_hi - w_lo
        def _zrow(r, _):
            def _zcol(k, _):
                rows0[r, pl.ds(k * 16, 16)] = jnp.zeros((16,), jnp.float32)
                return 0
            return lax.fori_loop(0, CW // 16, _zcol, 0)
        lax.fori_loop(0, WIN, _zrow, 0)
        for k in range(ROWS_PER_TILE // WIN):
            pltpu.sync_copy(rows0, spm.at[pl.ds(s * ROWS_PER_TILE + k * WIN, WIN)])
        plsc.subcore_barrier()

        def start(rbuf, slot, sem):
            pltpu.async_copy(tab.at[exr.at[slot, q]], rbuf, sem)

        def wait(rbuf, sem):
            pltpu.make_async_copy(tab.at[exr.at[0, q]], rbuf, sem).wait()

        def scatter(rbuf, slot):
            pltpu.sync_copy(rbuf, spm.at[exr.at[slot, NCHUNK]], add=True)

        pltpu.sync_copy(edges4.at[s, w_lo], exr.at[0])
        start(rows0, 0, gsem0)
        load_idx(w_lo + 1, 1)

        def gbody(g, _):
            b = w_lo + 2 * g + 2
            wait_idx()
            start(rows1, 1, gsem1)
            wait(rows0, gsem0)
            scatter(rows0, 0)
            load_idx(b, 0)
            wait_idx()
            start(rows0, 0, gsem0)
            wait(rows1, gsem1)
            scatter(rows1, 1)
            load_idx(b + 1, 1)
            return 0
        lax.fori_loop(0, nwin // 2 - 1, gbody, 0)
        wait_idx()
        start(rows1, 1, gsem1)
        wait(rows0, gsem0)
        scatter(rows0, 0)
        wait(rows1, gsem1)
        scatter(rows1, 1)

        plsc.subcore_barrier()
        pltpu.sync_copy(spm.at[pl.ds(s * ROWS_PER_TILE, ROWS_PER_TILE)],
                        out.at[pl.ds(s * ROWS_PER_TILE, ROWS_PER_TILE)])
        plsc.subcore_barrier()

    @pl.when(c == 0)
    def _():
        do_chunk(0, o0, 0, NWIN)
        do_chunk(1, o1, 0, NWIN)
        do_chunk(4, o4a, 0, NWIN // 2)

    @pl.when(c == 1)
    def _():
        do_chunk(2, o2, 0, NWIN)
        do_chunk(3, o3, 0, NWIN)
        do_chunk(4, o4b, NWIN // 2, NWIN)


def _sc_segment_sum(tab, edges4):
    mesh = plsc.VectorSubcoreMesh(core_axis_name="c", subcore_axis_name="s",
                                  num_cores=NC, num_subcores=NS)
    f = pl.kernel(
        _sc_segment_sum_body,
        out_type=[jax.ShapeDtypeStruct((NPAD, CW), jnp.float32)] * (NCHUNK + 1),
        mesh=mesh,
        scratch_types=[
            pltpu.VMEM_SHARED((NPAD, CW), jnp.float32),
            pltpu.VMEM((WIN, CW), jnp.float32),
            pltpu.VMEM((WIN, CW), jnp.float32),
            pltpu.VMEM((2, NCHUNK + 1, WIN), jnp.int32),
            pltpu.SemaphoreType.DMA,
            pltpu.SemaphoreType.DMA,
            pltpu.SemaphoreType.DMA,
        ],
    )
    return f(tab, edges4)


def _dot_t(a, w):
    return lax.dot_general(a, w, (((1,), (1,)), ((), ())),
                           preferred_element_type=jnp.float32)


def _xroot_body(x_ref, wroot_ref, brel_ref, out_ref):
    out_ref[...] = _dot_t(x_ref[...], wroot_ref[...]) + brel_ref[...]


def _xroot(cell_x, wroot, brel):
    blk = 2000
    return pl.pallas_call(
        _xroot_body,
        grid=(N // blk,),
        in_specs=[
            pl.BlockSpec((blk, D), lambda i: (i, 0)),
            pl.BlockSpec((D, D), lambda i: (0, 0)),
            pl.BlockSpec((1, D), lambda i: (0, 0)),
        ],
        out_specs=pl.BlockSpec((blk, D), lambda i: (i, 0)),
        out_shape=jax.ShapeDtypeStruct((N, D), jnp.float32),
    )(cell_x, wroot, brel)


def _tc_body(xr_ref, a0, a1, a2, a3, a4a, a4b, wrel_ref,
             gnw, gnb, gns, bng, bnb, linw, linb, out_ref,
             ssum, ssq, smax, smin):
    i = pl.program_id(0)

    h = xr_ref[...]
    for q, aq in enumerate((a0, a1, a2, a3)):
        h += _dot_t(aq[...], wrel_ref[:, q * CW:(q + 1) * CW])
    a4 = a4a[:, 0:2] + a4b[:, 0:2]
    h += _dot_t(a4, wrel_ref[:, 4 * CW:D])

    @pl.when(i == 0)
    def _():
        ssum[...] = jnp.zeros_like(ssum)
        ssq[...] = jnp.zeros_like(ssq)
        smax[...] = jnp.full_like(smax, -jnp.inf)
        smin[...] = jnp.full_like(smin, jnp.inf)

    ssum[...] += jnp.sum(h, axis=0, keepdims=True)
    ssq[...] += jnp.sum(h * h, axis=0, keepdims=True)
    smax[...] = jnp.maximum(smax[...], jnp.max(h, axis=0, keepdims=True))
    smin[...] = jnp.minimum(smin[...], jnp.min(h, axis=0, keepdims=True))

    @pl.when(i == NBLK - 1)
    def _():
        inv_n = 1.0 / N
        mean = ssum[...] * inv_n
        ex2 = ssq[...] * inv_n
        sm = gns[...] * mean
        gnvar = ex2 - 2.0 * sm * mean + sm * sm
        a1v = gnw[...] * lax.rsqrt(gnvar + 1e-5)
        b1v = gnb[...] - a1v * sm
        tvar = ex2 - mean * mean
        m1 = a1v * mean + b1v
        v1 = a1v * a1v * tvar
        a2v = bng[...] * lax.rsqrt(v1 + 1e-5)
        b2v = bnb[...] - a2v * m1
        A = a1v * a2v
        B = a2v * b1v + b2v
        r = jnp.where(A >= 0.0, A * smax[...], A * smin[...]) + B
        out_ref[...] = _dot_t(r, linw[...]) + linb[...]


def _tc_encode(xr, aggs, wrel, gnw, gnb, gns, bng, bnb, linw, linb):
    row_spec = pl.BlockSpec((BLK, D), lambda i: (i, 0))
    agg_spec = pl.BlockSpec((BLK, CW), lambda i: (i, 0))
    def whole(shape):
        return pl.BlockSpec(shape, lambda i: tuple(0 for _ in shape))
    return pl.pallas_call(
        _tc_body,
        grid=(NBLK,),
        in_specs=[
            row_spec, agg_spec, agg_spec, agg_spec, agg_spec, agg_spec,
            agg_spec,
            whole((D, D)),
            whole((1, D)), whole((1, D)), whole((1, D)),
            whole((1, D)), whole((1, D)),
            whole((DOUT, D)), whole((1, DOUT)),
        ],
        out_specs=whole((1, DOUT)),
        out_shape=jax.ShapeDtypeStruct((1, DOUT), jnp.float32),
        scratch_shapes=[
            pltpu.VMEM((1, D), jnp.float32),
            pltpu.VMEM((1, D), jnp.float32),
            pltpu.VMEM((1, D), jnp.float32),
            pltpu.VMEM((1, D), jnp.float32),
        ],
    )(xr, *aggs, wrel, gnw, gnb, gns, bng, bnb, linw, linb)


def kernel(cell_x, cell_edge_index, tissue_x, tissue_edge_index,
           assignment_mat, W_rel, b_rel, W_root, gn_weight, gn_bias,
           gn_mean_scale, bn_gamma, bn_beta, lin_W, lin_b):
    del tissue_x, tissue_edge_index, assignment_mat

    src = cell_edge_index[0].astype(jnp.int32)
    dst = cell_edge_index[1].astype(jnp.int32)

    npad_e = EPT_PAD - EPT
    tile_ids = jnp.arange(NS, dtype=jnp.int32)[:, None]
    j = jnp.arange(npad_e, dtype=jnp.int32)[None, :]
    pad_src = (tile_ids * 997 + j * 41) % N
    pad_dst = N + (j + tile_ids * 13) % (NPAD - N)
    src2 = jnp.concatenate([src.reshape(NS, EPT), pad_src], axis=1)
    dst2 = jnp.concatenate([dst.reshape(NS, EPT), pad_dst], axis=1)
    src2 = src2.reshape(NS, NWIN, 1, WIN)
    dst2 = dst2.reshape(NS, NWIN, 1, WIN)
    p = jnp.arange(NCHUNK + 1, dtype=jnp.int32).reshape(1, 1, NCHUNK + 1, 1)
    edges4 = jnp.where(p < NCHUNK, src2 + p * N, dst2)

    tab = jnp.concatenate(
        [cell_x[:, q * CW:(q + 1) * CW] for q in range(NCHUNK - 1)]
        + [jnp.pad(cell_x[:, (NCHUNK - 1) * CW:], ((0, 0), (0, DPAD - D)))],
        axis=0)

    aggs = _sc_segment_sum(tab, edges4)
    xr = _xroot(cell_x, W_root, b_rel.reshape(1, D))

    out = _tc_encode(
        xr, aggs, W_rel,
        gn_weight.reshape(1, D), gn_bias.reshape(1, D),
        gn_mean_scale.reshape(1, D), bn_gamma.reshape(1, D),
        bn_beta.reshape(1, D), lin_W, lin_b.reshape(1, DOUT))
    return out[:, None, :]


# speedup vs baseline: 4.5710x; 1.0003x over previous
import jax
import jax.numpy as jnp
from jax import lax
from jax.experimental import pallas as pl
from jax.experimental.pallas import tpu as pltpu
from jax.experimental.pallas import tpu_sc as plsc

N = 10000
E = 160000
D = 514
DOUT = 256
NPAD = 10240
CW = 128
NCHUNK = 5
DPAD = NCHUNK * CW

NC, NS = 2, 16
EPT = E // NS
WIN = 128
NWIN = 80
EPT_PAD = NWIN * WIN
ROWS_PER_TILE = NPAD // NS

NBLK = 5
BLK = N // NBLK


def _sc_segment_sum_body(tab, edges4, o0, o1, o2, o3, o4a, o4b,
                         spm, rows0, rows1, exr, gsem0, gsem1, isem):
    c = lax.axis_index("c")
    s = lax.axis_index("s")

    def load_idx(w, slot):
        pltpu.async_copy(edges4.at[s, w], exr.at[slot], isem)

    def wait_idx():
        pltpu.make_async_copy(edges4.at[s, 0], exr.at[0], isem).wait()

    def do_chunk(q, out, w_lo, w_hi):
        nwin = w_hi - w_lo
        def _zrow(r, _):
            def _zcol(k, _):
                rows0[r, pl.ds(k * 16, 16)] = jnp.zeros((16,), jnp.float32)
                return 0
            return lax.fori_loop(0, CW // 16, _zcol, 0)
        lax.fori_loop(0, WIN, _zrow, 0)
        for k in range(ROWS_PER_TILE // WIN):
            pltpu.sync_copy(rows0, spm.at[pl.ds(s * ROWS_PER_TILE + k * WIN, WIN)])
        plsc.subcore_barrier()

        def start(rbuf, slot, sem):
            pltpu.async_copy(tab.at[exr.at[slot, q]], rbuf, sem)

        def wait(rbuf, sem):
            pltpu.make_async_copy(tab.at[exr.at[0, q]], rbuf, sem).wait()

        def scatter(rbuf, slot):
            pltpu.sync_copy(rbuf, spm.at[exr.at[slot, NCHUNK]], add=True)

        pltpu.sync_copy(edges4.at[s, w_lo], exr.at[0])
        start(rows0, 0, gsem0)
        load_idx(w_lo + 1, 1)

        def gbody(g, _):
            b = w_lo + 2 * g + 2
            wait_idx()
            start(rows1, 1, gsem1)
            wait(rows0, gsem0)
            scatter(rows0, 0)
            load_idx(b, 0)
            wait_idx()
            start(rows0, 0, gsem0)
            wait(rows1, gsem1)
            scatter(rows1, 1)
            load_idx(b + 1, 1)
            return 0
        lax.fori_loop(0, nwin // 2 - 1, gbody, 0)
        wait_idx()
        start(rows1, 1, gsem1)
        wait(rows0, gsem0)
        scatter(rows0, 0)
        wait(rows1, gsem1)
        scatter(rows1, 1)

        plsc.subcore_barrier()
        pltpu.sync_copy(spm.at[pl.ds(s * ROWS_PER_TILE, ROWS_PER_TILE)],
                        out.at[pl.ds(s * ROWS_PER_TILE, ROWS_PER_TILE)])
        plsc.subcore_barrier()

    @pl.when(c == 0)
    def _():
        do_chunk(0, o0, 0, NWIN)
        do_chunk(1, o1, 0, NWIN)
        do_chunk(4, o4a, 0, NWIN // 2)

    @pl.when(c == 1)
    def _():
        do_chunk(2, o2, 0, NWIN)
        do_chunk(3, o3, 0, NWIN)
        do_chunk(4, o4b, NWIN // 2, NWIN)


def _sc_segment_sum(tab, edges4):
    mesh = plsc.VectorSubcoreMesh(core_axis_name="c", subcore_axis_name="s",
                                  num_cores=NC, num_subcores=NS)
    f = pl.kernel(
        _sc_segment_sum_body,
        out_type=[jax.ShapeDtypeStruct((NPAD, CW), jnp.float32)] * (NCHUNK + 1),
        mesh=mesh,
        scratch_types=[
            pltpu.VMEM_SHARED((NPAD, CW), jnp.float32),
            pltpu.VMEM((WIN, CW), jnp.float32),
            pltpu.VMEM((WIN, CW), jnp.float32),
            pltpu.VMEM((2, NCHUNK + 1, WIN), jnp.int32),
            pltpu.SemaphoreType.DMA,
            pltpu.SemaphoreType.DMA,
            pltpu.SemaphoreType.DMA,
        ],
    )
    return f(tab, edges4)


def _dot_t(a, w):
    return lax.dot_general(a, w, (((1,), (1,)), ((), ())),
                           preferred_element_type=jnp.float32)


def _xroot_body(x_ref, wroot_ref, brel_ref, out_ref):
    out_ref[...] = _dot_t(x_ref[...], wroot_ref[...]) + brel_ref[...]


def _xroot(cell_x, wroot, brel):
    blk = 2000
    return pl.pallas_call(
        _xroot_body,
        grid=(N // blk,),
        in_specs=[
            pl.BlockSpec((blk, D), lambda i: (i, 0)),
            pl.BlockSpec((D, D), lambda i: (0, 0)),
            pl.BlockSpec((1, D), lambda i: (0, 0)),
        ],
        out_specs=pl.BlockSpec((blk, D), lambda i: (i, 0)),
        out_shape=jax.ShapeDtypeStruct((N, D), jnp.float32),
    )(cell_x, wroot, brel)


def _tc_body(xr_ref, a0, a1, a2, a3, a4a, a4b, wrel_ref,
             gnw, gnb, gns, bng, bnb, linw, linb, out_ref,
             ssum, ssq, smax, smin):
    i = pl.program_id(0)

    h = xr_ref[...]
    for q, aq in enumerate((a0, a1, a2, a3)):
        h += _dot_t(aq[...], wrel_ref[:, q * CW:(q + 1) * CW])
    a4 = a4a[:, 0:2] + a4b[:, 0:2]
    h += _dot_t(a4, wrel_ref[:, 4 * CW:D])

    @pl.when(i == 0)
    def _():
        ssum[...] = jnp.zeros_like(ssum)
        ssq[...] = jnp.zeros_like(ssq)
        smax[...] = jnp.full_like(smax, -jnp.inf)
        smin[...] = jnp.full_like(smin, jnp.inf)

    ssum[...] += jnp.sum(h, axis=0, keepdims=True)
    ssq[...] += jnp.sum(h * h, axis=0, keepdims=True)
    smax[...] = jnp.maximum(smax[...], jnp.max(h, axis=0, keepdims=True))
    smin[...] = jnp.minimum(smin[...], jnp.min(h, axis=0, keepdims=True))

    @pl.when(i == NBLK - 1)
    def _():
        inv_n = 1.0 / N
        mean = ssum[...] * inv_n
        ex2 = ssq[...] * inv_n
        sm = gns[...] * mean
        gnvar = ex2 - 2.0 * sm * mean + sm * sm
        a1v = gnw[...] * lax.rsqrt(gnvar + 1e-5)
        b1v = gnb[...] - a1v * sm
        tvar = ex2 - mean * mean
        m1 = a1v * mean + b1v
        v1 = a1v * a1v * tvar
        a2v = bng[...] * lax.rsqrt(v1 + 1e-5)
        b2v = bnb[...] - a2v * m1
        A = a1v * a2v
        B = a2v * b1v + b2v
        r = jnp.where(A >= 0.0, A * smax[...], A * smin[...]) + B
        out_ref[...] = _dot_t(r, linw[...]) + linb[...]


def _tc_encode(xr, aggs, wrel, gnw, gnb, gns, bng, bnb, linw, linb):
    row_spec = pl.BlockSpec((BLK, D), lambda i: (i, 0))
    agg_spec = pl.BlockSpec((BLK, CW), lambda i: (i, 0))
    def whole(shape):
        return pl.BlockSpec(shape, lambda i: tuple(0 for _ in shape))
    return pl.pallas_call(
        _tc_body,
        grid=(NBLK,),
        in_specs=[
            row_spec, agg_spec, agg_spec, agg_spec, agg_spec, agg_spec,
            agg_spec,
            whole((D, D)),
            whole((1, D)), whole((1, D)), whole((1, D)),
            whole((1, D)), whole((1, D)),
            whole((DOUT, D)), whole((1, DOUT)),
        ],
        out_specs=whole((1, DOUT)),
        out_shape=jax.ShapeDtypeStruct((1, DOUT), jnp.float32),
        scratch_shapes=[
            pltpu.VMEM((1, D), jnp.float32),
            pltpu.VMEM((1, D), jnp.float32),
            pltpu.VMEM((1, D), jnp.float32),
            pltpu.VMEM((1, D), jnp.float32),
        ],
    )(xr, *aggs, wrel, gnw, gnb, gns, bng, bnb, linw, linb)


def kernel(cell_x, cell_edge_index, tissue_x, tissue_edge_index,
           assignment_mat, W_rel, b_rel, W_root, gn_weight, gn_bias,
           gn_mean_scale, bn_gamma, bn_beta, lin_W, lin_b):
    del tissue_x, tissue_edge_index, assignment_mat

    src = cell_edge_index[0].astype(jnp.int32)
    dst = cell_edge_index[1].astype(jnp.int32)

    npad_e = EPT_PAD - EPT
    tile_ids = jnp.arange(NS, dtype=jnp.int32)[:, None]
    j = jnp.arange(npad_e, dtype=jnp.int32)[None, :]
    pad_src = (tile_ids * 997 + j * 41) % N
    pad_dst = N + (j + tile_ids * 13) % (NPAD - N)
    src2 = jnp.concatenate([src.reshape(NS, EPT), pad_src], axis=1)
    dst2 = jnp.concatenate([dst.reshape(NS, EPT), pad_dst], axis=1)
    src2 = src2.reshape(NS, NWIN, 1, WIN)
    dst2 = dst2.reshape(NS, NWIN, 1, WIN)
    p = jnp.arange(NCHUNK + 1, dtype=jnp.int32).reshape(1, 1, NCHUNK + 1, 1)
    edges4 = jnp.where(p < NCHUNK, src2 + p * N, dst2)

    tab = jnp.concatenate(
        [cell_x[:, q * CW:(q + 1) * CW] for q in range(NCHUNK - 1)]
        + [jnp.pad(cell_x[:, (NCHUNK - 1) * CW:], ((0, 0), (0, DPAD - D)))],
        axis=0)

    aggs = _sc_segment_sum(tab, edges4)
    xr = _xroot(cell_x, W_root, b_rel.reshape(1, D))

    out = _tc_encode(
        xr, aggs, W_rel,
        gn_weight.reshape(1, D), gn_bias.reshape(1, D),
        gn_mean_scale.reshape(1, D), bn_gamma.reshape(1, D),
        bn_beta.reshape(1, D), lin_W, lin_b.reshape(1, DOUT))
    return out[:, None, :]
